# baseline (device time: 8156 ns/iter reference)
import jax
import jax.numpy as jnp
from jax import lax
from jax.experimental import pallas as pl
from jax.experimental.pallas import tpu as pltpu


def kernel(x):
    m, n = x.shape
    bm = 128
    nblk = m // bm

    def body(x_ref, out_ref, pack_ref, comm_ref, send_sem, recv_sem):
        i = pl.program_id(0)
        my_x = lax.axis_index("x")
        my_y = lax.axis_index("y")
        peer = (my_x, 1 - my_y)
        barrier_sem = pltpu.get_barrier_semaphore()

        @pl.when(i == 0)
        def _():
            pl.semaphore_signal(
                barrier_sem, inc=1, device_id=peer,
                device_id_type=pl.DeviceIdType.MESH,
            )

        out_ref[pl.ds(i * bm, bm), :] = jnp.max(
            x_ref[:, :], axis=1, keepdims=True
        )

        @pl.when(i == nblk - 1)
        def _():
            pl.semaphore_wait(barrier_sem, 1)
            rdma = pltpu.make_async_remote_copy(
                src_ref=pack_ref,
                dst_ref=comm_ref,
                send_sem=send_sem,
                recv_sem=recv_sem,
                device_id=peer,
                device_id_type=pl.DeviceIdType.MESH,
            )
            rdma.start()
            rdma.wait()
            out_ref[pl.ds(0, 1), :] = jnp.maximum(
                out_ref[0:1, :], comm_ref[0:1, 0:1]
            )

    return pl.pallas_call(
        body,
        grid=(nblk,),
        out_shape=jax.ShapeDtypeStruct((m, 1), x.dtype),
        in_specs=[pl.BlockSpec((bm, n), lambda i: (i, 0))],
        out_specs=pl.BlockSpec((m, 1), lambda i: (0, 0)),
        scratch_shapes=[
            pltpu.VMEM((nblk, 128), x.dtype),
            pltpu.VMEM((nblk, 128), x.dtype),
            pltpu.SemaphoreType.DMA,
            pltpu.SemaphoreType.DMA,
        ],
        compiler_params=pltpu.CompilerParams(collective_id=0),
    )(x)


# device time: 8068 ns/iter; 1.0109x vs baseline; 1.0109x over previous
import jax
import jax.numpy as jnp
from jax import lax
from jax.experimental import pallas as pl
from jax.experimental.pallas import tpu as pltpu


def kernel(x):
    m, n = x.shape
    bm = 128
    nblk = m // bm
    ncol = n // 128

    def body(x_ref, out_ref, pack_ref, comm_ref, send_sem, recv_sem):
        i = pl.program_id(0)
        my_x = lax.axis_index("x")
        my_y = lax.axis_index("y")
        peer = (my_x, 1 - my_y)
        barrier_sem = pltpu.get_barrier_semaphore()

        @pl.when(i == 0)
        def _():
            pl.semaphore_signal(
                barrier_sem, inc=1, device_id=peer,
                device_id_type=pl.DeviceIdType.MESH,
            )

        xb = x_ref[:, :]
        y = xb[:, 0:128]
        for k in range(1, ncol):
            y = jnp.maximum(y, xb[:, k * 128:(k + 1) * 128])
        p = jnp.max(y.T, axis=0, keepdims=True)
        pack_ref[pl.ds(i, 1), :] = p

        @pl.when(i == nblk - 1)
        def _():
            pl.semaphore_wait(barrier_sem, 1)
            rdma = pltpu.make_async_remote_copy(
                src_ref=pack_ref,
                dst_ref=comm_ref,
                send_sem=send_sem,
                recv_sem=recv_sem,
                device_id=peer,
                device_id_type=pl.DeviceIdType.MESH,
            )
            rdma.start()
            rdma.wait()
            out_ref[:, :] = jnp.maximum(pack_ref[:, :], comm_ref[:, :])

    out = pl.pallas_call(
        body,
        grid=(nblk,),
        out_shape=jax.ShapeDtypeStruct((nblk, 128), x.dtype),
        in_specs=[pl.BlockSpec((bm, n), lambda i: (i, 0))],
        out_specs=pl.BlockSpec((nblk, 128), lambda i: (0, 0)),
        scratch_shapes=[
            pltpu.VMEM((nblk, 128), x.dtype),
            pltpu.VMEM((nblk, 128), x.dtype),
            pltpu.SemaphoreType.DMA,
            pltpu.SemaphoreType.DMA,
        ],
        compiler_params=pltpu.CompilerParams(collective_id=0),
    )(x)
    return out.reshape(m, 1)


# device time: 7415 ns/iter; 1.0999x vs baseline; 1.0881x over previous
import jax
import jax.numpy as jnp
from jax import lax
from jax.experimental import pallas as pl
from jax.experimental.pallas import tpu as pltpu


def kernel(x):
    m, n = x.shape
    bm = 256
    nblk = m // bm
    ncol = n // 128

    def body(x_ref, out_ref, pack_ref, comm_ref, send_sem, recv_sem):
        i = pl.program_id(0)
        my_x = lax.axis_index("x")
        my_y = lax.axis_index("y")
        peer = (my_x, 1 - my_y)
        barrier_sem = pltpu.get_barrier_semaphore()

        @pl.when(i == 0)
        def _():
            pl.semaphore_signal(
                barrier_sem, inc=1, device_id=peer,
                device_id_type=pl.DeviceIdType.MESH,
            )

        xb = x_ref[:, :]
        y = xb[:, 0:128]
        for k in range(1, ncol):
            y = jnp.maximum(y, xb[:, k * 128:(k + 1) * 128])
        p = jnp.max(y.T, axis=0, keepdims=True)
        pack_ref[pl.ds(i, 1), :] = p

        @pl.when(i == nblk - 1)
        def _():
            pl.semaphore_wait(barrier_sem, 1)
            rdma = pltpu.make_async_remote_copy(
                src_ref=pack_ref,
                dst_ref=comm_ref,
                send_sem=send_sem,
                recv_sem=recv_sem,
                device_id=peer,
                device_id_type=pl.DeviceIdType.MESH,
            )
            rdma.start()
            rdma.wait()
            out_ref[:, :] = jnp.maximum(pack_ref[:, :], comm_ref[:, :])

    out = pl.pallas_call(
        body,
        grid=(nblk,),
        out_shape=jax.ShapeDtypeStruct((nblk, bm), x.dtype),
        in_specs=[pl.BlockSpec((bm, n), lambda i: (i, 0))],
        out_specs=pl.BlockSpec((nblk, bm), lambda i: (0, 0)),
        scratch_shapes=[
            pltpu.VMEM((nblk, bm), x.dtype),
            pltpu.VMEM((nblk, bm), x.dtype),
            pltpu.SemaphoreType.DMA,
            pltpu.SemaphoreType.DMA,
        ],
        compiler_params=pltpu.CompilerParams(collective_id=0),
    )(x)
    return out.reshape(m, 1)


# device time: 6839 ns/iter; 1.1926x vs baseline; 1.0842x over previous
import jax
import jax.numpy as jnp
from jax import lax
from jax.experimental import pallas as pl
from jax.experimental.pallas import tpu as pltpu


def kernel(x):
    m, n = x.shape
    bm = 1536
    nblk = m // bm
    ncol = n // 128

    def body(x_ref, out_ref, pack_ref, comm_ref, send_sem, recv_sem):
        i = pl.program_id(0)
        my_x = lax.axis_index("x")
        my_y = lax.axis_index("y")
        peer = (my_x, 1 - my_y)
        barrier_sem = pltpu.get_barrier_semaphore()

        @pl.when(i == 0)
        def _():
            pl.semaphore_signal(
                barrier_sem, inc=1, device_id=peer,
                device_id_type=pl.DeviceIdType.MESH,
            )

        xb = x_ref[:, :]
        y = xb[:, 0:128]
        for k in range(1, ncol):
            y = jnp.maximum(y, xb[:, k * 128:(k + 1) * 128])
        p = jnp.max(y.T, axis=0, keepdims=True)
        pack_ref[pl.ds(i, 1), :] = p

        @pl.when(i == nblk - 1)
        def _():
            pl.semaphore_wait(barrier_sem, 1)
            rdma = pltpu.make_async_remote_copy(
                src_ref=pack_ref,
                dst_ref=comm_ref,
                send_sem=send_sem,
                recv_sem=recv_sem,
                device_id=peer,
                device_id_type=pl.DeviceIdType.MESH,
            )
            rdma.start()
            rdma.wait()
            out_ref[:, :] = jnp.maximum(pack_ref[:, :], comm_ref[:, :])

    out = pl.pallas_call(
        body,
        grid=(nblk,),
        out_shape=jax.ShapeDtypeStruct((nblk, bm), x.dtype),
        in_specs=[pl.BlockSpec((bm, n), lambda i: (i, 0))],
        out_specs=pl.BlockSpec((nblk, bm), lambda i: (0, 0)),
        scratch_shapes=[
            pltpu.VMEM((nblk, bm), x.dtype),
            pltpu.VMEM((nblk, bm), x.dtype),
            pltpu.SemaphoreType.DMA,
            pltpu.SemaphoreType.DMA,
        ],
        compiler_params=pltpu.CompilerParams(collective_id=0),
    )(x)
    return out.reshape(m, 1)
